# baseline (speedup 1.0000x reference)
import functools

import jax
import jax.numpy as jnp
from jax import lax
from jax.experimental import pallas as pl
from jax.experimental.pallas import tpu as pltpu
from jax.experimental.pallas import tpu_sc as plsc


def _gather_kernel(batch, dim, n_workers, nc):
    b_per_w = batch // n_workers
    ch = 128
    n_ch = b_per_w // ch
    groups = ch // 16
    per = 128 // dim
    shift = per.bit_length() - 1
    mesh = plsc.VectorSubcoreMesh(core_axis_name="c", subcore_axis_name="s")

    @functools.partial(
        pl.kernel,
        mesh=mesh,
        compiler_params=pltpu.CompilerParams(needs_layout_passes=False),
        out_type=jax.ShapeDtypeStruct((dim, batch), jnp.float32),
        scratch_types=[
            pltpu.VMEM((b_per_w,), jnp.int32),
            pltpu.VMEM((b_per_w,), jnp.int32),
            pltpu.VMEM((ch, per * dim), jnp.float32),
            pltpu.VMEM((dim, b_per_w), jnp.float32),
            pltpu.SemaphoreType.DMA,
        ],
    )
    def k(idx_hbm, table_hbm, out_hbm, idx_v, rs_v, rows_v, cols_v, sem):
        wid = lax.axis_index("s") * nc + lax.axis_index("c")
        base = wid * b_per_w
        pltpu.sync_copy(idx_hbm.at[pl.ds(base, b_per_w)], idx_v)

        def rs_body(g, _):
            iv = idx_v[pl.ds(g * 16, 16)]
            rs_v[pl.ds(g * 16, 16)] = lax.shift_right_logical(
                iv, jnp.full((16,), shift, jnp.int32)
            )
            return _

        lax.fori_loop(0, b_per_w // 16, rs_body, 0)

        for c in range(n_ch):
            pltpu.async_copy(
                table_hbm.at[rs_v.at[pl.ds(c * ch, ch)]], rows_v, sem
            ).wait()

            def sel_body(g, _):
                i0 = c * ch + g * 16
                iv = idx_v[pl.ds(i0, 16)]
                colb = lax.bitwise_and(
                    iv, jnp.full((16,), per - 1, jnp.int32)
                ) * jnp.full((16,), dim, jnp.int32)
                row16 = lax.iota(jnp.int32, 16) + jnp.full(
                    (16,), g * 16, jnp.int32
                )
                for d in range(dim):
                    v = plsc.load_gather(
                        rows_v, [row16, colb + jnp.full((16,), d, jnp.int32)]
                    )
                    cols_v[d, pl.ds(i0, 16)] = v
                return _

            lax.fori_loop(0, groups, sel_body, 0)

        pltpu.sync_copy(cols_v, out_hbm.at[:, pl.ds(base, b_per_w)])

    return k


def kernel(node_idx, emb):
    info = plsc.get_sparse_core_info()
    nw = info.num_cores * info.num_subcores
    batch = node_idx.shape[0]
    n_nodes, dim = emb.shape
    per = 128 // dim
    table2 = emb.reshape(n_nodes // per, per * dim)
    k = _gather_kernel(batch, dim, nw, info.num_cores)
    out_t = k(node_idx.astype(jnp.int32), table2)
    return jnp.transpose(out_t)


# speedup vs baseline: 1.6763x; 1.6763x over previous
import functools

import jax
import jax.numpy as jnp
from jax import lax
from jax.experimental import pallas as pl
from jax.experimental.pallas import tpu as pltpu
from jax.experimental.pallas import tpu_sc as plsc


def _gather_kernel(batch, dim, n_workers, nc):
    b_per_w = batch // n_workers
    n_groups = b_per_w // 16
    mesh = plsc.VectorSubcoreMesh(core_axis_name="c", subcore_axis_name="s")

    @functools.partial(
        pl.kernel,
        mesh=mesh,
        out_type=jax.ShapeDtypeStruct((batch, dim), jnp.float32),
        scratch_types=[
            pltpu.VMEM((b_per_w,), jnp.int32),
            pltpu.VMEM((b_per_w, dim), jnp.float32),
            pltpu.SemaphoreType.DMA,
        ],
    )
    def k(idx_hbm, table_hbm, out_hbm, idx_v, rows_v, sem):
        wid = lax.axis_index("s") * nc + lax.axis_index("c")
        base = wid * b_per_w
        pltpu.sync_copy(idx_hbm.at[pl.ds(base, b_per_w)], idx_v)

        def grp_body(g, _):
            iv = idx_v[pl.ds(g * 16, 16)]
            for r in range(16):
                pltpu.make_async_copy(
                    table_hbm.at[iv[r]], rows_v.at[g * 16 + r], sem
                ).start()
            return _

        lax.fori_loop(0, n_groups, grp_body, 0)

        def drain_body(j, _):
            pltpu.make_async_copy(table_hbm.at[0], rows_v.at[0], sem).wait()
            return _

        lax.fori_loop(0, b_per_w, drain_body, 0)
        pltpu.sync_copy(rows_v, out_hbm.at[pl.ds(base, b_per_w)])

    return k


def kernel(node_idx, emb):
    info = plsc.get_sparse_core_info()
    nw = info.num_cores * info.num_subcores
    batch = node_idx.shape[0]
    dim = emb.shape[1]
    k = _gather_kernel(batch, dim, nw, info.num_cores)
    return k(node_idx.astype(jnp.int32), emb)
